# baseline (speedup 1.0000x reference)
import jax
import jax.numpy as jnp
from jax.experimental import pallas as pl
from jax.experimental.pallas import tpu as pltpu

ALPHA = 1.702
LIMIT = 7.0
FFB = 1024


def _moe_kernel(x_ref, gw_ref, gb_ref, wg_ref, bg_ref, wu_ref, bu_ref,
                wd_ref, bd_ref, out_ref, wrout_ref, xbf_ref):
    e = pl.program_id(0)
    f = pl.program_id(1)
    T, E = wrout_ref.shape

    @pl.when((e == 0) & (f == 0))
    def _router():
        x = x_ref[...]
        xbf_ref[...] = x.astype(jnp.bfloat16)
        logits = jax.lax.dot_general(
            x, gw_ref[...], (((1,), (1,)), ((), ())),
            precision=jax.lax.Precision.HIGHEST,
            preferred_element_type=jnp.float32) + gb_ref[...]
        lane = jax.lax.broadcasted_iota(jnp.int32, (T, E), 1)
        m1 = jnp.max(logits, axis=1, keepdims=True)
        a1 = jnp.min(jnp.where(logits == m1, lane, E), axis=1, keepdims=True)
        masked = jnp.where(lane == a1, -jnp.inf, logits)
        m2 = jnp.max(masked, axis=1, keepdims=True)
        a2 = jnp.min(jnp.where(masked == m2, lane, E), axis=1, keepdims=True)
        e2 = jnp.exp(m2 - m1)
        denom = 1.0 + e2
        w1 = 1.0 / denom
        w2 = e2 / denom
        wrout_ref[...] = (w1 * (lane == a1) + w2 * (lane == a2)).astype(jnp.float32)

    lane = jax.lax.broadcasted_iota(jnp.int32, (T, E), 1)
    we = jnp.sum(wrout_ref[...] * (lane == e), axis=1, keepdims=True)

    xbf = xbf_ref[...]
    g = jnp.dot(xbf, wg_ref[0].astype(jnp.bfloat16),
                preferred_element_type=jnp.float32) + bg_ref[0, 0]
    u = jnp.dot(xbf, wu_ref[0].astype(jnp.bfloat16),
                preferred_element_type=jnp.float32) + bu_ref[0, 0]
    g = jnp.minimum(g, LIMIT)
    u = jnp.clip(u, -LIMIT, LIMIT)
    glu = g * jax.nn.sigmoid(ALPHA * g)
    gated = (u + 1.0) * glu * we
    partial = jnp.dot(gated.astype(jnp.bfloat16), wd_ref[0].astype(jnp.bfloat16),
                      preferred_element_type=jnp.float32)
    delta = partial + (f == 0).astype(jnp.float32) * (we * bd_ref[0, 0])

    @pl.when((e == 0) & (f == 0))
    def _init():
        out_ref[...] = delta

    @pl.when((e > 0) | (f > 0))
    def _acc():
        out_ref[...] += delta


@jax.jit
def kernel(hidden_states, gate_w, gate_b, Wg, bg, Wu, bu, Wd, bd):
    T, H = hidden_states.shape
    E, _, FF = Wg.shape
    nf = FF // FFB
    return pl.pallas_call(
        _moe_kernel,
        grid=(E, nf),
        in_specs=[
            pl.BlockSpec((T, H), lambda e, f: (0, 0)),
            pl.BlockSpec((E, H), lambda e, f: (0, 0)),
            pl.BlockSpec((1, E), lambda e, f: (0, 0)),
            pl.BlockSpec((1, H, FFB), lambda e, f: (e, 0, f)),
            pl.BlockSpec((1, 1, FFB), lambda e, f: (e, 0, f)),
            pl.BlockSpec((1, H, FFB), lambda e, f: (e, 0, f)),
            pl.BlockSpec((1, 1, FFB), lambda e, f: (e, 0, f)),
            pl.BlockSpec((1, FFB, H), lambda e, f: (e, f, 0)),
            pl.BlockSpec((1, 1, H), lambda e, f: (e, 0, 0)),
        ],
        out_specs=pl.BlockSpec((T, H), lambda e, f: (0, 0)),
        out_shape=jax.ShapeDtypeStruct((T, H), jnp.float32),
        scratch_shapes=[
            pltpu.VMEM((T, E), jnp.float32),
            pltpu.VMEM((T, H), jnp.bfloat16),
        ],
    )(hidden_states, gate_w, gate_b.reshape(1, E), Wg, bg.reshape(E, 1, FF),
      Wu, bu.reshape(E, 1, FF), Wd, bd.reshape(E, 1, H))


# speedup vs baseline: 1.0157x; 1.0157x over previous
import jax
import jax.numpy as jnp
from jax.experimental import pallas as pl
from jax.experimental.pallas import tpu as pltpu

ALPHA = 1.702
LIMIT = 7.0
FFB = 512


def _moe_kernel(x_ref, gw_ref, gb_ref, wg_ref, bg_ref, wu_ref, bu_ref,
                wd_ref, bd_ref, out_ref, wrout_ref, xbf_ref):
    e = pl.program_id(0)
    f = pl.program_id(1)
    T, E = wrout_ref.shape

    @pl.when((e == 0) & (f == 0))
    def _router():
        x = x_ref[...]
        xbf_ref[...] = x.astype(jnp.bfloat16)
        logits = jax.lax.dot_general(
            x, gw_ref[...], (((1,), (1,)), ((), ())),
            precision=jax.lax.Precision.HIGHEST,
            preferred_element_type=jnp.float32) + gb_ref[...]
        lane = jax.lax.broadcasted_iota(jnp.int32, (T, E), 1)
        m1 = jnp.max(logits, axis=1, keepdims=True)
        a1 = jnp.min(jnp.where(logits == m1, lane, E), axis=1, keepdims=True)
        masked = jnp.where(lane == a1, -jnp.inf, logits)
        m2 = jnp.max(masked, axis=1, keepdims=True)
        a2 = jnp.min(jnp.where(masked == m2, lane, E), axis=1, keepdims=True)
        e2 = jnp.exp(m2 - m1)
        denom = 1.0 + e2
        w1 = 1.0 / denom
        w2 = e2 / denom
        wrout_ref[...] = (w1 * (lane == a1) + w2 * (lane == a2)).astype(jnp.float32)

    lane = jax.lax.broadcasted_iota(jnp.int32, (T, E), 1)
    we = jnp.sum(wrout_ref[...] * (lane == e), axis=1, keepdims=True)

    xbf = xbf_ref[...]
    g = jnp.dot(xbf, wg_ref[0].astype(jnp.bfloat16),
                preferred_element_type=jnp.float32) + bg_ref[0, 0]
    u = jnp.dot(xbf, wu_ref[0].astype(jnp.bfloat16),
                preferred_element_type=jnp.float32) + bu_ref[0, 0]
    g = jnp.minimum(g, LIMIT)
    u = jnp.clip(u, -LIMIT, LIMIT)
    glu = g * jax.nn.sigmoid(ALPHA * g)
    gated = (u + 1.0) * glu * we
    partial = jnp.dot(gated.astype(jnp.bfloat16), wd_ref[0].astype(jnp.bfloat16),
                      preferred_element_type=jnp.float32)

    @pl.when((e == 0) & (f == 0))
    def _init():
        out_ref[...] = partial + we * bd_ref[0, 0]

    @pl.when((e > 0) & (f == 0))
    def _bias():
        out_ref[...] += partial + we * bd_ref[0, 0]

    @pl.when(f > 0)
    def _acc():
        out_ref[...] += partial


@jax.jit
def kernel(hidden_states, gate_w, gate_b, Wg, bg, Wu, bu, Wd, bd):
    T, H = hidden_states.shape
    E, _, FF = Wg.shape
    nf = FF // FFB
    return pl.pallas_call(
        _moe_kernel,
        grid=(E, nf),
        in_specs=[
            pl.BlockSpec((T, H), lambda e, f: (0, 0)),
            pl.BlockSpec((E, H), lambda e, f: (0, 0)),
            pl.BlockSpec((1, E), lambda e, f: (0, 0)),
            pl.BlockSpec((1, H, FFB), lambda e, f: (e, 0, f)),
            pl.BlockSpec((1, 1, FFB), lambda e, f: (e, 0, f)),
            pl.BlockSpec((1, H, FFB), lambda e, f: (e, 0, f)),
            pl.BlockSpec((1, 1, FFB), lambda e, f: (e, 0, f)),
            pl.BlockSpec((1, FFB, H), lambda e, f: (e, f, 0)),
            pl.BlockSpec((1, 1, H), lambda e, f: (e, 0, 0)),
        ],
        out_specs=pl.BlockSpec((T, H), lambda e, f: (0, 0)),
        out_shape=jax.ShapeDtypeStruct((T, H), jnp.float32),
        scratch_shapes=[
            pltpu.VMEM((T, E), jnp.float32),
            pltpu.VMEM((T, H), jnp.bfloat16),
        ],
    )(hidden_states, gate_w, gate_b.reshape(1, E), Wg, bg.reshape(E, 1, FF),
      Wu, bu.reshape(E, 1, FF), Wd, bd.reshape(E, 1, H))


# speedup vs baseline: 1.0194x; 1.0037x over previous
import jax
import jax.numpy as jnp
from jax.experimental import pallas as pl
from jax.experimental.pallas import tpu as pltpu

ALPHA = 1.702
LIMIT = 7.0
FFB = 512


def _moe_kernel(x_ref, gw_ref, gb_ref, wg_ref, bg_ref, wu_ref, bu_ref,
                wd_ref, bd_ref, out_ref, wrout_ref, xbf_ref):
    e = pl.program_id(0)
    f = pl.program_id(1)
    T, E = wrout_ref.shape

    @pl.when((e == 0) & (f == 0))
    def _router():
        x = x_ref[...]
        xbf_ref[...] = x.astype(jnp.bfloat16)
        logits = jax.lax.dot_general(
            x.astype(jnp.bfloat16), gw_ref[...].astype(jnp.bfloat16),
            (((1,), (1,)), ((), ())),
            preferred_element_type=jnp.float32) + gb_ref[...]
        lane = jax.lax.broadcasted_iota(jnp.int32, (T, E), 1)
        m1 = jnp.max(logits, axis=1, keepdims=True)
        a1 = jnp.min(jnp.where(logits == m1, lane, E), axis=1, keepdims=True)
        masked = jnp.where(lane == a1, -jnp.inf, logits)
        m2 = jnp.max(masked, axis=1, keepdims=True)
        a2 = jnp.min(jnp.where(masked == m2, lane, E), axis=1, keepdims=True)
        e2 = jnp.exp(m2 - m1)
        denom = 1.0 + e2
        w1 = 1.0 / denom
        w2 = e2 / denom
        wrout_ref[...] = (w1 * (lane == a1) + w2 * (lane == a2)).astype(jnp.float32)

    lane = jax.lax.broadcasted_iota(jnp.int32, (T, E), 1)
    we = jnp.sum(wrout_ref[...] * (lane == e), axis=1, keepdims=True)

    xbf = xbf_ref[...]
    g = jnp.dot(xbf, wg_ref[0].astype(jnp.bfloat16),
                preferred_element_type=jnp.float32) + bg_ref[0, 0]
    u = jnp.dot(xbf, wu_ref[0].astype(jnp.bfloat16),
                preferred_element_type=jnp.float32) + bu_ref[0, 0]
    g = jnp.minimum(g, LIMIT)
    u = jnp.clip(u, -LIMIT, LIMIT)
    glu = g * jax.nn.sigmoid(ALPHA * g)
    gated = (u + 1.0) * glu * we
    partial = jnp.dot(gated.astype(jnp.bfloat16), wd_ref[0].astype(jnp.bfloat16),
                      preferred_element_type=jnp.float32)

    @pl.when((e == 0) & (f == 0))
    def _init():
        out_ref[...] = partial + we * bd_ref[0, 0]

    @pl.when((e > 0) & (f == 0))
    def _bias():
        out_ref[...] += partial + we * bd_ref[0, 0]

    @pl.when(f > 0)
    def _acc():
        out_ref[...] += partial


@jax.jit
def kernel(hidden_states, gate_w, gate_b, Wg, bg, Wu, bu, Wd, bd):
    T, H = hidden_states.shape
    E, _, FF = Wg.shape
    nf = FF // FFB
    return pl.pallas_call(
        _moe_kernel,
        grid=(E, nf),
        in_specs=[
            pl.BlockSpec((T, H), lambda e, f: (0, 0)),
            pl.BlockSpec((E, H), lambda e, f: (0, 0)),
            pl.BlockSpec((1, E), lambda e, f: (0, 0)),
            pl.BlockSpec((1, H, FFB), lambda e, f: (e, 0, f)),
            pl.BlockSpec((1, 1, FFB), lambda e, f: (e, 0, f)),
            pl.BlockSpec((1, H, FFB), lambda e, f: (e, 0, f)),
            pl.BlockSpec((1, 1, FFB), lambda e, f: (e, 0, f)),
            pl.BlockSpec((1, FFB, H), lambda e, f: (e, f, 0)),
            pl.BlockSpec((1, 1, H), lambda e, f: (e, 0, 0)),
        ],
        out_specs=pl.BlockSpec((T, H), lambda e, f: (0, 0)),
        out_shape=jax.ShapeDtypeStruct((T, H), jnp.float32),
        scratch_shapes=[
            pltpu.VMEM((T, E), jnp.float32),
            pltpu.VMEM((T, H), jnp.bfloat16),
        ],
    )(hidden_states, gate_w, gate_b.reshape(1, E), Wg, bg.reshape(E, 1, FF),
      Wu, bu.reshape(E, 1, FF), Wd, bd.reshape(E, 1, H))
